# baseline (speedup 1.0000x reference)
import functools

import jax
import jax.numpy as jnp
from jax import lax
from jax.experimental import pallas as pl
from jax.experimental.pallas import tpu as pltpu
from jax.experimental.pallas import tpu_sc as plsc

B = 1024
L = 200
EMB = 16
NC = 2
NS = 16
NW = NC * NS
ROWS_PER_W = B // NW
GATHER_PER_W = ROWS_PER_W * L
CHUNK = 128
NCHUNK = GATHER_PER_W // CHUNK


def _sc_body(seq_hbm, items_hbm, table_hbm, u_hbm, v_hbm,
             idx_v, rows_v, usum_v, itm_v, vrows_v, sem, sem2):
    wid = lax.axis_index("s") * NC + lax.axis_index("c")
    base = wid * ROWS_PER_W

    pltpu.sync_copy(seq_hbm.at[wid], idx_v)

    def issue(j, c):
        pltpu.async_copy(table_hbm.at[idx_v.at[j]],
                         rows_v.at[pl.ds(j * CHUNK, CHUNK)], sem)
        return c
    lax.fori_loop(0, NCHUNK, issue, 0)

    pltpu.sync_copy(items_hbm.at[wid], itm_v)
    pltpu.async_copy(table_hbm.at[itm_v], vrows_v, sem2).wait()
    pltpu.sync_copy(vrows_v, v_hbm.at[pl.ds(base, ROWS_PER_W)])

    def drain(j, c):
        pltpu.make_async_copy(table_hbm.at[idx_v.at[0]],
                              rows_v.at[pl.ds(0, CHUNK)], sem).wait()
        return c
    lax.fori_loop(0, NCHUNK, drain, 0)

    def row(r, c):
        def inner(i, acc):
            b0 = r * L + i * 8
            for k in range(8):
                acc = acc + rows_v[b0 + k]
            return acc
        acc = lax.fori_loop(0, L // 8, inner,
                            jnp.zeros((EMB,), jnp.float32))
        usum_v[r] = acc
        return c
    lax.fori_loop(0, ROWS_PER_W, row, 0)
    pltpu.sync_copy(usum_v, u_hbm.at[pl.ds(base, ROWS_PER_W)])


_sc_gather = pl.kernel(
    _sc_body,
    out_type=[jax.ShapeDtypeStruct((B, EMB), jnp.float32),
              jax.ShapeDtypeStruct((B, EMB), jnp.float32)],
    mesh=plsc.VectorSubcoreMesh(core_axis_name="c", subcore_axis_name="s"),
    scratch_types=[
        pltpu.VMEM((NCHUNK, CHUNK), jnp.int32),
        pltpu.VMEM((GATHER_PER_W, EMB), jnp.float32),
        pltpu.VMEM((ROWS_PER_W, EMB), jnp.float32),
        pltpu.VMEM((ROWS_PER_W,), jnp.int32),
        pltpu.VMEM((ROWS_PER_W, EMB), jnp.float32),
        pltpu.SemaphoreType.DMA,
        pltpu.SemaphoreType.DMA,
    ],
    compiler_params=pltpu.CompilerParams(use_tc_tiling_on_sc=False),
)


def _tc_body(u_ref, v_ref, o_ref):
    u = u_ref[...]
    v = v_ref[...]
    un = u * lax.rsqrt(jnp.maximum(jnp.sum(u * u, axis=1, keepdims=True),
                                   1e-24))
    vn = v * lax.rsqrt(jnp.maximum(jnp.sum(v * v, axis=1, keepdims=True),
                                   1e-24))
    o_ref[...] = lax.dot_general(un, vn, (((1,), (1,)), ((), ())),
                                 preferred_element_type=jnp.float32)


_tc_score = pl.pallas_call(
    _tc_body,
    out_shape=jax.ShapeDtypeStruct((B, B), jnp.float32),
)


@jax.jit
def kernel(input_seqs, items_to_predict, table):
    seq_r = input_seqs.reshape(NW, NCHUNK, CHUNK)
    items_r = items_to_predict[:, -1].reshape(NW, ROWS_PER_W)
    u_sum, v_rows = _sc_gather(seq_r, items_r, table)
    scores = _tc_score(u_sum, v_rows)
    return scores.reshape(B, B, 1)
